# Initial kernel scaffold; baseline (speedup 1.0000x reference)
import jax
import jax.numpy as jnp
from jax.experimental import pallas as pl


def kernel(x_gate, x_experts, noise, Wg, Wn, We, be):
    raise NotImplementedError("write your pallas kernel here")


# speedup vs baseline: 2.3929x; 2.3929x over previous
import functools

import jax
import jax.numpy as jnp
from jax.experimental import pallas as pl
from jax.experimental.pallas import tpu as pltpu

_NEG = -1e30


def _moe_dense_body(xg_ref, xe_ref, nz_ref, wg_ref, wn_ref, we_ref, be_ref,
                    out_ref, gates_ref):
    e = pl.program_id(1)
    n_e = pl.num_programs(1)

    @pl.when(e == 0)
    def _gating():
        xg = xg_ref[...]
        dn = (((1,), (1,)), ((), ()))
        clean = jax.lax.dot_general(xg, wg_ref[...], dn,
                                    preferred_element_type=jnp.float32)
        raw = jax.lax.dot_general(xg, wn_ref[...], dn,
                                  preferred_element_type=jnp.float32)
        sp = jnp.maximum(raw, 0.0) + jnp.log1p(jnp.exp(-jnp.abs(raw)))
        noisy = clean + nz_ref[...] * sp
        cols = jax.lax.broadcasted_iota(jnp.int32, noisy.shape, 1)
        m1 = jnp.max(noisy, axis=1, keepdims=True)
        i1 = jnp.min(jnp.where(noisy == m1, cols, noisy.shape[1]),
                     axis=1, keepdims=True)
        mask1 = cols == i1
        masked = jnp.where(mask1, _NEG, noisy)
        m2 = jnp.max(masked, axis=1, keepdims=True)
        i2 = jnp.min(jnp.where(masked == m2, cols, noisy.shape[1]),
                     axis=1, keepdims=True)
        z = jnp.exp(m2 - m1)
        w1 = 1.0 / (1.0 + z)
        w2 = z / (1.0 + z)
        gates_ref[...] = jnp.where(mask1, w1, 0.0) + jnp.where(cols == i2, w2, 0.0)

    x = xe_ref[...]
    y = jax.lax.dot_general(x, we_ref[0], (((1,), (1,)), ((), ())),
                            preferred_element_type=jnp.float32)
    g = jnp.sum(gates_ref[...] * (jax.lax.broadcasted_iota(
        jnp.int32, gates_ref.shape, 1) == e), axis=1, keepdims=True)
    contrib = (y + be_ref[0]) * g

    @pl.when(e == 0)
    def _init():
        out_ref[...] = contrib

    @pl.when(e > 0)
    def _acc():
        out_ref[...] += contrib


def _moe_dense(x_gate, x_experts, noise, Wg, Wn, We, be, tt, interpret=False):
    n, d = x_gate.shape
    e = Wg.shape[0]
    grid = (n // tt, e)
    return pl.pallas_call(
        _moe_dense_body,
        grid=grid,
        in_specs=[
            pl.BlockSpec((tt, d), lambda t, j: (t, 0)),
            pl.BlockSpec((tt, d), lambda t, j: (t, 0)),
            pl.BlockSpec((tt, e), lambda t, j: (t, 0)),
            pl.BlockSpec((e, d), lambda t, j: (0, 0)),
            pl.BlockSpec((e, d), lambda t, j: (0, 0)),
            pl.BlockSpec((1, d, d), lambda t, j: (j, 0, 0)),
            pl.BlockSpec((1, 1, d), lambda t, j: (j, 0, 0)),
        ],
        out_specs=pl.BlockSpec((tt, d), lambda t, j: (t, 0)),
        out_shape=jax.ShapeDtypeStruct((n, d), jnp.float32),
        scratch_shapes=[pltpu.VMEM((tt, e), jnp.float32)],
        interpret=interpret,
    )(x_gate, x_experts, noise, Wg, Wn, We, be.reshape(e, 1, d))


def kernel(x_gate, x_experts, noise, Wg, Wn, We, be):
    n = x_gate.shape[0]
    tt = min(n, 1024)
    return _moe_dense(x_gate, x_experts, noise, Wg, Wn, We, be, tt)
